# baseline (speedup 1.0000x reference)
import functools

import jax
import jax.numpy as jnp
import numpy as np
from jax import lax
from jax.experimental import pallas as pl
from jax.experimental.pallas import tpu as pltpu
from jax.experimental.pallas import tpu_sc as plsc

N = 10000
E = 160000
D = 256
HD = 128
NB = 64
NT = 16
CH = 128
NCH = 80
EP = NT * NCH * CH
ACC = 10240
NTRASH = 240
SLAB = ACC // NT
BN = 2000
GRID = N // BN

_mesh = plsc.VectorSubcoreMesh(core_axis_name="c", subcore_axis_name="s")


@functools.partial(
    pl.kernel,
    mesh=_mesh,
    out_type=[
        jax.ShapeDtypeStruct((ACC, HD), jnp.float32),
        jax.ShapeDtypeStruct((ACC, HD), jnp.float32),
    ],
    scratch_types=[
        pltpu.VMEM((NCH // 2, CH), jnp.int32),
        pltpu.VMEM((NCH // 2, CH), jnp.int32),
        pltpu.VMEM((CH, HD), jnp.float32),
        pltpu.VMEM((CH, HD), jnp.float32),
        pltpu.VMEM_SHARED((ACC, HD), jnp.float32),
        pltpu.SemaphoreType.DMA,
        pltpu.SemaphoreType.DMA,
    ],
)
def _sc_segment_sum(tab0, tab1, srcr, dstr, zeros, out0, out1,
                    src_v, dst_v, buf0, buf1, acc, sem0, sem1):
    c = lax.axis_index("c")
    s = lax.axis_index("s")
    pltpu.sync_copy(zeros.at[pl.ds(s * SLAB, SLAB)], acc.at[pl.ds(s * SLAB, SLAB)])
    plsc.subcore_barrier()

    PCH = NCH // 2
    NPAIR = PCH // 2

    def run(tab):
        def gather(j, buf, sem):
            return pltpu.async_copy(tab.at[src_v.at[j]], buf, sem)

        for ph in range(2):
            pltpu.sync_copy(srcr.at[s, pl.ds(ph * PCH, PCH)], src_v)
            pltpu.sync_copy(dstr.at[s, pl.ds(ph * PCH, PCH)], dst_v)
            gather(0, buf0, sem0)

            def body(p, carry):
                a = 2 * p
                b = a + 1
                gather(b, buf1, sem1)
                pltpu.make_async_copy(tab.at[src_v.at[a]], buf0, sem0).wait()
                pltpu.sync_copy(buf0, acc.at[dst_v.at[a]], add=True)

                @pl.when(p < NPAIR - 1)
                def _():
                    gather(a + 2, buf0, sem0)

                pltpu.make_async_copy(tab.at[src_v.at[b]], buf1, sem1).wait()
                pltpu.sync_copy(buf1, acc.at[dst_v.at[b]], add=True)
                return carry

            lax.fori_loop(0, NPAIR, body, 0)

    pl.when(c == 0)(lambda: run(tab0))
    pl.when(c == 1)(lambda: run(tab1))
    plsc.subcore_barrier()

    def copy_out(out):
        pltpu.sync_copy(acc.at[pl.ds(s * SLAB, SLAB)], out.at[pl.ds(s * SLAB, SLAB)])

    pl.when(c == 0)(lambda: copy_out(out0))
    pl.when(c == 1)(lambda: copy_out(out1))


def _mm_stats_body(a0, a1, s0, s1, w, b, t_ref, st_ref, acc):
    i = pl.program_id(0)
    pre = jnp.concatenate([a0[...] + s0[...], a1[...] + s1[...]], axis=1)
    t = lax.dot_general(pre, w[...], (((1,), (1,)), ((), ())),
                        preferred_element_type=jnp.float32) + b[...]
    t_ref[...] = t
    blk = jnp.concatenate([jnp.sum(t, axis=0, keepdims=True),
                           jnp.sum(t * t, axis=0, keepdims=True)], axis=0)

    @pl.when(i == 0)
    def _():
        acc[...] = blk

    @pl.when(i > 0)
    def _():
        acc[...] = acc[...] + blk

    @pl.when(i == pl.num_programs(0) - 1)
    def _():
        st_ref[...] = acc[...]


def _mm_stats(a0, a1, s0, s1, w, b):
    return pl.pallas_call(
        _mm_stats_body,
        grid=(GRID,),
        in_specs=[
            pl.BlockSpec((BN, HD), lambda i: (i, 0)),
            pl.BlockSpec((BN, HD), lambda i: (i, 0)),
            pl.BlockSpec((BN, HD), lambda i: (i, 0)),
            pl.BlockSpec((BN, HD), lambda i: (i, 0)),
            pl.BlockSpec((D, D), lambda i: (0, 0)),
            pl.BlockSpec((1, D), lambda i: (0, 0)),
        ],
        out_specs=[
            pl.BlockSpec((BN, D), lambda i: (i, 0)),
            pl.BlockSpec((2, D), lambda i: (0, 0)),
        ],
        out_shape=[
            jax.ShapeDtypeStruct((N, D), jnp.float32),
            jax.ShapeDtypeStruct((2, D), jnp.float32),
        ],
        scratch_shapes=[pltpu.VMEM((2, D), jnp.float32)],
    )(a0, a1, s0, s1, w, b)


def _bn(t, st, g, be):
    m = st[0:1, :] * (1.0 / N)
    v = st[1:2, :] * (1.0 / N) - m * m
    return (t - m) * lax.rsqrt(v + 1e-5) * g + be


def _bn_mm_body(t_ref, st_ref, g_ref, be_ref, w_ref, b_ref, h0_ref, h1_ref):
    h = jnp.maximum(_bn(t_ref[...], st_ref[...], g_ref[...], be_ref[...]), 0.0)
    h = lax.dot_general(h, w_ref[...], (((1,), (1,)), ((), ())),
                        preferred_element_type=jnp.float32) + b_ref[...]
    h = jnp.maximum(h, 0.0)
    h0_ref[...] = h[:, :HD]
    h1_ref[...] = h[:, HD:]


def _bn_mm(t, st, g, be, w, b):
    return pl.pallas_call(
        _bn_mm_body,
        grid=(GRID,),
        in_specs=[
            pl.BlockSpec((BN, D), lambda i: (i, 0)),
            pl.BlockSpec((2, D), lambda i: (0, 0)),
            pl.BlockSpec((1, D), lambda i: (0, 0)),
            pl.BlockSpec((1, D), lambda i: (0, 0)),
            pl.BlockSpec((D, D), lambda i: (0, 0)),
            pl.BlockSpec((1, D), lambda i: (0, 0)),
        ],
        out_specs=[
            pl.BlockSpec((BN, HD), lambda i: (i, 0)),
            pl.BlockSpec((BN, HD), lambda i: (i, 0)),
        ],
        out_shape=[
            jax.ShapeDtypeStruct((N, HD), jnp.float32),
            jax.ShapeDtypeStruct((N, HD), jnp.float32),
        ],
    )(t, st, g, be, w, b)


def _bn_pool_body(t_ref, st_ref, g_ref, be_ref, bid_ref, out_ref, acc):
    i = pl.program_id(0)
    h2 = jnp.maximum(_bn(t_ref[...], st_ref[...], g_ref[...], be_ref[...]), 0.0)
    bid = bid_ref[0, 0, :]
    onehot = (bid[:, None] == lax.broadcasted_iota(jnp.int32, (BN, NB), 1)
              ).astype(jnp.float32)
    blk = lax.dot_general(onehot, h2, (((0,), (0,)), ((), ())),
                          preferred_element_type=jnp.float32)

    @pl.when(i == 0)
    def _():
        acc[...] = blk

    @pl.when(i > 0)
    def _():
        acc[...] = acc[...] + blk

    @pl.when(i == pl.num_programs(0) - 1)
    def _():
        out_ref[...] = acc[...]


def _bn_pool(t, st, g, be, bid):
    return pl.pallas_call(
        _bn_pool_body,
        grid=(GRID,),
        in_specs=[
            pl.BlockSpec((BN, D), lambda i: (i, 0)),
            pl.BlockSpec((2, D), lambda i: (0, 0)),
            pl.BlockSpec((1, D), lambda i: (0, 0)),
            pl.BlockSpec((1, D), lambda i: (0, 0)),
            pl.BlockSpec((1, 1, BN), lambda i: (i, 0, 0)),
        ],
        out_specs=pl.BlockSpec((NB, D), lambda i: (0, 0)),
        out_shape=jax.ShapeDtypeStruct((NB, D), jnp.float32),
        scratch_shapes=[pltpu.VMEM((NB, D), jnp.float32)],
    )(t, st, g, be, bid)


def _spectral_normalize(*Ws):
    def init(W):
        o = W.shape[0]
        u0 = jnp.full((1, o), 1.0 / np.sqrt(o), jnp.float32)
        return (u0, lax.dot_general(u0, W, (((1,), (0,)), ((), ()))))

    def step(W, u, v):
        v = lax.dot_general(u, W, (((1,), (0,)), ((), ())))
        v = v / (jnp.sqrt(jnp.sum(v * v)) + 1e-12)
        u = lax.dot_general(v, W, (((1,), (1,)), ((), ())))
        u = u / (jnp.sqrt(jnp.sum(u * u)) + 1e-12)
        return (u, v)

    def body(_, uvs):
        return tuple(step(W, u, v) for W, (u, v) in zip(Ws, uvs))

    uvs = lax.fori_loop(0, 30, body, tuple(init(W) for W in Ws))
    out = []
    for W, (u, v) in zip(Ws, uvs):
        wv = lax.dot_general(v, W, (((1,), (1,)), ((), ())))
        out.append(W / jnp.sum(u * wv))
    return out


def _sn_topo_body(topo_ref, wt_ref, bt_ref, wf1_ref, wf2_ref,
                  gt_ref, wf1n_ref, wf2n_ref):
    wt, wf1, wf2 = _spectral_normalize(wt_ref[...], wf1_ref[...], wf2_ref[...])
    gt_ref[...] = jnp.maximum(
        lax.dot_general(topo_ref[...], wt, (((1,), (1,)), ((), ())),
                        preferred_element_type=jnp.float32) + bt_ref[...], 0.0)
    wf1n_ref[...] = wf1
    wf2n_ref[...] = wf2


def _sn_topo(topo, wt, bt, wf1, wf2):
    return pl.pallas_call(
        _sn_topo_body,
        out_shape=[
            jax.ShapeDtypeStruct((NB, D), jnp.float32),
            jax.ShapeDtypeStruct(wf1.shape, jnp.float32),
            jax.ShapeDtypeStruct(wf2.shape, jnp.float32),
        ],
    )(topo, wt, bt, wf1, wf2)


def _head_body(gs_ref, gt_ref, wf1_ref, bf1_ref, wf2_ref, bf2_ref, out_ref):
    gcat = jnp.concatenate([gs_ref[...], gt_ref[...]], axis=1)
    z = lax.dot_general(gcat, wf1_ref[...], (((1,), (1,)), ((), ())),
                        preferred_element_type=jnp.float32) + bf1_ref[...]
    z = jnp.where(z > 0, z, jnp.exp(jnp.minimum(z, 0.0)) - 1.0)
    out_ref[...] = lax.dot_general(z, wf2_ref[...], (((1,), (1,)), ((), ())),
                                   preferred_element_type=jnp.float32) + bf2_ref[...]


def _head(gs, gt, wf1n, bf1, wf2n, bf2):
    return pl.pallas_call(
        _head_body,
        out_shape=jax.ShapeDtypeStruct((NB, 16), jnp.float32),
    )(gs, gt, wf1n, bf1, wf2n, bf2)


def kernel(x, edge_index, batch, topo_vec, W1a, b1a, g1, be1, W1b, b1b,
           W2a, b2a, g2, be2, Wt, bt, Wf1, bf1, Wf2, bf2):
    f32 = jnp.float32
    x0 = x[:, :HD]
    x1 = x[:, HD:]
    src = edge_index[0].astype(jnp.int32)
    dst = edge_index[1].astype(jnp.int32)
    srcr = jnp.concatenate([src, jnp.zeros((EP - E,), jnp.int32)]).reshape(NT, NCH, CH)
    pad_dst = N + jnp.arange(EP - E, dtype=jnp.int32) % NTRASH
    dstr = jnp.concatenate([dst, pad_dst]).reshape(NT, NCH, CH)
    zeros = jnp.zeros((ACC, HD), f32)

    gt, wf1n, wf2n = _sn_topo(topo_vec, Wt, bt.reshape(1, -1), Wf1, Wf2)
    seg0, seg1 = _sc_segment_sum(x0, x1, srcr, dstr, zeros)
    t1, st1 = _mm_stats(x0, x1, seg0, seg1, W1a, b1a.reshape(1, -1))
    h0, h1 = _bn_mm(t1, st1, g1.reshape(1, -1), be1.reshape(1, -1),
                    W1b, b1b.reshape(1, -1))
    s20, s21 = _sc_segment_sum(h0, h1, srcr, dstr, zeros)
    t2, st2 = _mm_stats(h0, h1, s20, s21, W2a, b2a.reshape(1, -1))
    gs = _bn_pool(t2, st2, g2.reshape(1, -1), be2.reshape(1, -1),
                  batch.astype(jnp.int32).reshape(GRID, 1, BN))
    return _head(gs, gt, wf1n, bf1.reshape(1, -1), wf2n, bf2.reshape(1, -1))


# speedup vs baseline: 1.0783x; 1.0783x over previous
import functools

import jax
import jax.numpy as jnp
import numpy as np
from jax import lax
from jax.experimental import pallas as pl
from jax.experimental.pallas import tpu as pltpu
from jax.experimental.pallas import tpu_sc as plsc

N = 10000
E = 160000
D = 256
HD = 128
NB = 64
NT = 16
CH = 128
NCH = 79
EP = NT * NCH * CH
ACC = 10240
NTRASH = 240
SLAB = ACC // NT
BN = 2000
GRID = N // BN

_mesh = plsc.VectorSubcoreMesh(core_axis_name="c", subcore_axis_name="s")


@functools.partial(
    pl.kernel,
    mesh=_mesh,
    out_type=[
        jax.ShapeDtypeStruct((ACC, HD), jnp.float32),
        jax.ShapeDtypeStruct((ACC, HD), jnp.float32),
    ],
    scratch_types=[
        pltpu.VMEM((NCH, CH), jnp.int32),
        pltpu.VMEM((NCH, CH), jnp.int32),
        pltpu.VMEM((CH, HD), jnp.float32),
        pltpu.VMEM_SHARED((ACC, HD), jnp.float32),
        pltpu.SemaphoreType.DMA,
    ],
)
def _sc_segment_sum(tab, srcr, dstr, out0, out1,
                    src_v, dst_v, rows_v, acc, sem):
    c = lax.axis_index("c")
    s = lax.axis_index("s")
    zf = jnp.zeros((16,), jnp.float32)

    def zrow(r, carry):
        for k in range(HD // 16):
            rows_v[r, pl.ds(k * 16, 16)] = zf
        return carry

    lax.fori_loop(0, CH, zrow, 0)
    for k in range(SLAB // CH):
        pltpu.sync_copy(rows_v, acc.at[pl.ds(s * SLAB + k * CH, CH)])
    pltpu.sync_copy(srcr.at[c, s], src_v)
    pltpu.sync_copy(dstr.at[s], dst_v)
    plsc.subcore_barrier()

    def body(j, carry):
        pltpu.async_copy(tab.at[src_v.at[j]], rows_v, sem).wait()
        pltpu.sync_copy(rows_v, acc.at[dst_v.at[j]], add=True)
        return carry

    lax.fori_loop(0, NCH, body, 0)
    plsc.subcore_barrier()

    def copy_out(out):
        pltpu.sync_copy(acc.at[pl.ds(s * SLAB, SLAB)], out.at[pl.ds(s * SLAB, SLAB)])

    pl.when(c == 0)(lambda: copy_out(out0))
    pl.when(c == 1)(lambda: copy_out(out1))


def _mm_stats_body(a, s0, s1, w, b, t_ref, st_ref, acc):
    i = pl.program_id(0)
    pre = a[...] + jnp.concatenate([s0[...], s1[...]], axis=1)
    t = lax.dot_general(pre, w[...], (((1,), (1,)), ((), ())),
                        preferred_element_type=jnp.float32) + b[...]
    t_ref[...] = t
    blk = jnp.concatenate([jnp.sum(t, axis=0, keepdims=True),
                           jnp.sum(t * t, axis=0, keepdims=True)], axis=0)

    @pl.when(i == 0)
    def _():
        acc[...] = blk

    @pl.when(i > 0)
    def _():
        acc[...] = acc[...] + blk

    @pl.when(i == pl.num_programs(0) - 1)
    def _():
        st_ref[...] = acc[...]


def _mm_stats(a, s0, s1, w, b):
    return pl.pallas_call(
        _mm_stats_body,
        grid=(GRID,),
        in_specs=[
            pl.BlockSpec((BN, D), lambda i: (i, 0)),
            pl.BlockSpec((BN, HD), lambda i: (i, 0)),
            pl.BlockSpec((BN, HD), lambda i: (i, 0)),
            pl.BlockSpec((D, D), lambda i: (0, 0)),
            pl.BlockSpec((1, D), lambda i: (0, 0)),
        ],
        out_specs=[
            pl.BlockSpec((BN, D), lambda i: (i, 0)),
            pl.BlockSpec((2, D), lambda i: (0, 0)),
        ],
        out_shape=[
            jax.ShapeDtypeStruct((N, D), jnp.float32),
            jax.ShapeDtypeStruct((2, D), jnp.float32),
        ],
        scratch_shapes=[pltpu.VMEM((2, D), jnp.float32)],
    )(a, s0, s1, w, b)


def _bn(t, st, g, be):
    m = st[0:1, :] * (1.0 / N)
    v = st[1:2, :] * (1.0 / N) - m * m
    return (t - m) * lax.rsqrt(v + 1e-5) * g + be


def _bn_mm_body(t_ref, st_ref, g_ref, be_ref, w_ref, b_ref, h_ref):
    h = jnp.maximum(_bn(t_ref[...], st_ref[...], g_ref[...], be_ref[...]), 0.0)
    h = lax.dot_general(h, w_ref[...], (((1,), (1,)), ((), ())),
                        preferred_element_type=jnp.float32) + b_ref[...]
    h_ref[...] = jnp.maximum(h, 0.0)


def _bn_mm(t, st, g, be, w, b):
    return pl.pallas_call(
        _bn_mm_body,
        grid=(GRID,),
        in_specs=[
            pl.BlockSpec((BN, D), lambda i: (i, 0)),
            pl.BlockSpec((2, D), lambda i: (0, 0)),
            pl.BlockSpec((1, D), lambda i: (0, 0)),
            pl.BlockSpec((1, D), lambda i: (0, 0)),
            pl.BlockSpec((D, D), lambda i: (0, 0)),
            pl.BlockSpec((1, D), lambda i: (0, 0)),
        ],
        out_specs=pl.BlockSpec((BN, D), lambda i: (i, 0)),
        out_shape=jax.ShapeDtypeStruct((N, D), jnp.float32),
    )(t, st, g, be, w, b)


def _bn_pool_body(t_ref, st_ref, g_ref, be_ref, bid_ref, out_ref, acc):
    i = pl.program_id(0)
    h2 = jnp.maximum(_bn(t_ref[...], st_ref[...], g_ref[...], be_ref[...]), 0.0)
    bid = bid_ref[0, 0, :]
    onehot = (bid[:, None] == lax.broadcasted_iota(jnp.int32, (BN, NB), 1)
              ).astype(jnp.float32)
    blk = lax.dot_general(onehot, h2, (((0,), (0,)), ((), ())),
                          preferred_element_type=jnp.float32)

    @pl.when(i == 0)
    def _():
        acc[...] = blk

    @pl.when(i > 0)
    def _():
        acc[...] = acc[...] + blk

    @pl.when(i == pl.num_programs(0) - 1)
    def _():
        out_ref[...] = acc[...]


def _bn_pool(t, st, g, be, bid):
    return pl.pallas_call(
        _bn_pool_body,
        grid=(GRID,),
        in_specs=[
            pl.BlockSpec((BN, D), lambda i: (i, 0)),
            pl.BlockSpec((2, D), lambda i: (0, 0)),
            pl.BlockSpec((1, D), lambda i: (0, 0)),
            pl.BlockSpec((1, D), lambda i: (0, 0)),
            pl.BlockSpec((1, 1, BN), lambda i: (i, 0, 0)),
        ],
        out_specs=pl.BlockSpec((NB, D), lambda i: (0, 0)),
        out_shape=jax.ShapeDtypeStruct((NB, D), jnp.float32),
        scratch_shapes=[pltpu.VMEM((NB, D), jnp.float32)],
    )(t, st, g, be, bid)


def _spectral_normalize(*Ws):
    def init(W):
        o = W.shape[0]
        u0 = jnp.full((1, o), 1.0 / np.sqrt(o), jnp.float32)
        return (u0, lax.dot_general(u0, W, (((1,), (0,)), ((), ()))))

    def step(W, u, v):
        v = lax.dot_general(u, W, (((1,), (0,)), ((), ())))
        v = v / (jnp.sqrt(jnp.sum(v * v)) + 1e-12)
        u = lax.dot_general(v, W, (((1,), (1,)), ((), ())))
        u = u / (jnp.sqrt(jnp.sum(u * u)) + 1e-12)
        return (u, v)

    def body(_, uvs):
        return tuple(step(W, u, v) for W, (u, v) in zip(Ws, uvs))

    uvs = lax.fori_loop(0, 30, body, tuple(init(W) for W in Ws))
    out = []
    for W, (u, v) in zip(Ws, uvs):
        wv = lax.dot_general(v, W, (((1,), (1,)), ((), ())))
        out.append(W / jnp.sum(u * wv))
    return out


def _sn_topo_body(topo_ref, wt_ref, bt_ref, wf1_ref, wf2_ref,
                  gt_ref, wf1n_ref, wf2n_ref):
    wt, wf1, wf2 = _spectral_normalize(wt_ref[...], wf1_ref[...], wf2_ref[...])
    gt_ref[...] = jnp.maximum(
        lax.dot_general(topo_ref[...], wt, (((1,), (1,)), ((), ())),
                        preferred_element_type=jnp.float32) + bt_ref[...], 0.0)
    wf1n_ref[...] = wf1
    wf2n_ref[...] = wf2


def _sn_topo(topo, wt, bt, wf1, wf2):
    return pl.pallas_call(
        _sn_topo_body,
        out_shape=[
            jax.ShapeDtypeStruct((NB, D), jnp.float32),
            jax.ShapeDtypeStruct(wf1.shape, jnp.float32),
            jax.ShapeDtypeStruct(wf2.shape, jnp.float32),
        ],
    )(topo, wt, bt, wf1, wf2)


def _head_body(gs_ref, gt_ref, wf1_ref, bf1_ref, wf2_ref, bf2_ref, out_ref):
    gcat = jnp.concatenate([gs_ref[...], gt_ref[...]], axis=1)
    z = lax.dot_general(gcat, wf1_ref[...], (((1,), (1,)), ((), ())),
                        preferred_element_type=jnp.float32) + bf1_ref[...]
    z = jnp.where(z > 0, z, jnp.exp(jnp.minimum(z, 0.0)) - 1.0)
    out_ref[...] = lax.dot_general(z, wf2_ref[...], (((1,), (1,)), ((), ())),
                                   preferred_element_type=jnp.float32) + bf2_ref[...]


def _head(gs, gt, wf1n, bf1, wf2n, bf2):
    return pl.pallas_call(
        _head_body,
        out_shape=jax.ShapeDtypeStruct((NB, 16), jnp.float32),
    )(gs, gt, wf1n, bf1, wf2n, bf2)


def kernel(x, edge_index, batch, topo_vec, W1a, b1a, g1, be1, W1b, b1b,
           W2a, b2a, g2, be2, Wt, bt, Wf1, bf1, Wf2, bf2):
    src = edge_index[0].astype(jnp.int32)
    dst = edge_index[1].astype(jnp.int32)
    src_p = jnp.concatenate([src, jnp.zeros((EP - E,), jnp.int32)])
    srcr = jnp.stack([2 * src_p, 2 * src_p + 1]).reshape(2, NT, NCH, CH)
    pad_dst = N + jnp.arange(EP - E, dtype=jnp.int32) % NTRASH
    dstr = jnp.concatenate([dst, pad_dst]).reshape(NT, NCH, CH)

    gt, wf1n, wf2n = _sn_topo(topo_vec, Wt, bt.reshape(1, -1), Wf1, Wf2)
    seg0, seg1 = _sc_segment_sum(x.reshape(2 * N, HD), srcr, dstr)
    t1, st1 = _mm_stats(x, seg0, seg1, W1a, b1a.reshape(1, -1))
    h = _bn_mm(t1, st1, g1.reshape(1, -1), be1.reshape(1, -1),
               W1b, b1b.reshape(1, -1))
    s20, s21 = _sc_segment_sum(h.reshape(2 * N, HD), srcr, dstr)
    t2, st2 = _mm_stats(h, s20, s21, W2a, b2a.reshape(1, -1))
    gs = _bn_pool(t2, st2, g2.reshape(1, -1), be2.reshape(1, -1),
                  batch.astype(jnp.int32).reshape(GRID, 1, BN))
    return _head(gs, gt, wf1n, bf1.reshape(1, -1), wf2n, bf2.reshape(1, -1))


# speedup vs baseline: 1.1249x; 1.0432x over previous
import functools

import jax
import jax.numpy as jnp
import numpy as np
from jax import lax
from jax.experimental import pallas as pl
from jax.experimental.pallas import tpu as pltpu
from jax.experimental.pallas import tpu_sc as plsc

N = 10000
E = 160000
D = 256
HD = 128
NB = 64
NT = 16
CH = 128
NCH = 79
EP = NT * NCH * CH
ACC = 10240
NTRASH = 240
SLAB = ACC // NT
BN = 2000
GRID = N // BN

_mesh = plsc.VectorSubcoreMesh(core_axis_name="c", subcore_axis_name="s")


@functools.partial(
    pl.kernel,
    mesh=_mesh,
    out_type=[
        jax.ShapeDtypeStruct((ACC, HD), jnp.float32),
        jax.ShapeDtypeStruct((ACC, HD), jnp.float32),
    ],
    scratch_types=[
        pltpu.VMEM((NCH, CH), jnp.int32),
        pltpu.VMEM((NCH, CH), jnp.int32),
        pltpu.VMEM((CH, HD), jnp.float32),
        pltpu.VMEM_SHARED((ACC, HD), jnp.float32),
        pltpu.SemaphoreType.DMA,
    ],
)
def _sc_segment_sum(tab0, tab1, srcr, dstr, out0, out1,
                    src_v, dst_v, rows_v, acc, sem):
    c = lax.axis_index("c")
    s = lax.axis_index("s")
    zf = jnp.zeros((16,), jnp.float32)

    def zrow(r, carry):
        for k in range(HD // 16):
            rows_v[r, pl.ds(k * 16, 16)] = zf
        return carry

    lax.fori_loop(0, CH, zrow, 0)
    for k in range(SLAB // CH):
        pltpu.sync_copy(rows_v, acc.at[pl.ds(s * SLAB + k * CH, CH)])
    pltpu.sync_copy(srcr.at[s], src_v)
    pltpu.sync_copy(dstr.at[s], dst_v)
    plsc.subcore_barrier()

    def run(tab):
        def body(j, carry):
            pltpu.async_copy(tab.at[src_v.at[j]], rows_v, sem).wait()
            pltpu.sync_copy(rows_v, acc.at[dst_v.at[j]], add=True)
            return carry
        lax.fori_loop(0, NCH, body, 0)

    pl.when(c == 0)(lambda: run(tab0))
    pl.when(c == 1)(lambda: run(tab1))
    plsc.subcore_barrier()

    def copy_out(out):
        pltpu.sync_copy(acc.at[pl.ds(s * SLAB, SLAB)], out.at[pl.ds(s * SLAB, SLAB)])

    pl.when(c == 0)(lambda: copy_out(out0))
    pl.when(c == 1)(lambda: copy_out(out1))


def _mm_stats_body(a0, a1, s0, s1, w, b, t_ref, st_ref, acc):
    i = pl.program_id(0)
    pre = jnp.concatenate([a0[...] + s0[...], a1[...] + s1[...]], axis=1)
    t = lax.dot_general(pre, w[...], (((1,), (1,)), ((), ())),
                        preferred_element_type=jnp.float32) + b[...]
    t_ref[...] = t
    blk = jnp.concatenate([jnp.sum(t, axis=0, keepdims=True),
                           jnp.sum(t * t, axis=0, keepdims=True)], axis=0)

    @pl.when(i == 0)
    def _():
        acc[...] = blk

    @pl.when(i > 0)
    def _():
        acc[...] = acc[...] + blk

    @pl.when(i == pl.num_programs(0) - 1)
    def _():
        st_ref[...] = acc[...]


def _mm_stats(a0, a1, s0, s1, w, b):
    return pl.pallas_call(
        _mm_stats_body,
        grid=(GRID,),
        in_specs=[
            pl.BlockSpec((BN, HD), lambda i: (i, 0)),
            pl.BlockSpec((BN, HD), lambda i: (i, 0)),
            pl.BlockSpec((BN, HD), lambda i: (i, 0)),
            pl.BlockSpec((BN, HD), lambda i: (i, 0)),
            pl.BlockSpec((D, D), lambda i: (0, 0)),
            pl.BlockSpec((1, D), lambda i: (0, 0)),
        ],
        out_specs=[
            pl.BlockSpec((BN, D), lambda i: (i, 0)),
            pl.BlockSpec((2, D), lambda i: (0, 0)),
        ],
        out_shape=[
            jax.ShapeDtypeStruct((N, D), jnp.float32),
            jax.ShapeDtypeStruct((2, D), jnp.float32),
        ],
        scratch_shapes=[pltpu.VMEM((2, D), jnp.float32)],
    )(a0, a1, s0, s1, w, b)


def _bn(t, st, g, be):
    m = st[0:1, :] * (1.0 / N)
    v = st[1:2, :] * (1.0 / N) - m * m
    return (t - m) * lax.rsqrt(v + 1e-5) * g + be


def _bn_mm_body(t_ref, st_ref, g_ref, be_ref, w_ref, b_ref, h0_ref, h1_ref):
    h = jnp.maximum(_bn(t_ref[...], st_ref[...], g_ref[...], be_ref[...]), 0.0)
    h = lax.dot_general(h, w_ref[...], (((1,), (1,)), ((), ())),
                        preferred_element_type=jnp.float32) + b_ref[...]
    h = jnp.maximum(h, 0.0)
    h0_ref[...] = h[:, :HD]
    h1_ref[...] = h[:, HD:]


def _bn_mm(t, st, g, be, w, b):
    return pl.pallas_call(
        _bn_mm_body,
        grid=(GRID,),
        in_specs=[
            pl.BlockSpec((BN, D), lambda i: (i, 0)),
            pl.BlockSpec((2, D), lambda i: (0, 0)),
            pl.BlockSpec((1, D), lambda i: (0, 0)),
            pl.BlockSpec((1, D), lambda i: (0, 0)),
            pl.BlockSpec((D, D), lambda i: (0, 0)),
            pl.BlockSpec((1, D), lambda i: (0, 0)),
        ],
        out_specs=[
            pl.BlockSpec((BN, HD), lambda i: (i, 0)),
            pl.BlockSpec((BN, HD), lambda i: (i, 0)),
        ],
        out_shape=[
            jax.ShapeDtypeStruct((N, HD), jnp.float32),
            jax.ShapeDtypeStruct((N, HD), jnp.float32),
        ],
    )(t, st, g, be, w, b)


def _bn_pool_body(t_ref, st_ref, g_ref, be_ref, bid_ref, out_ref, acc):
    i = pl.program_id(0)
    h2 = jnp.maximum(_bn(t_ref[...], st_ref[...], g_ref[...], be_ref[...]), 0.0)
    bid = bid_ref[0, 0, :]
    onehot = (bid[:, None] == lax.broadcasted_iota(jnp.int32, (BN, NB), 1)
              ).astype(jnp.float32)
    blk = lax.dot_general(onehot, h2, (((0,), (0,)), ((), ())),
                          preferred_element_type=jnp.float32)

    @pl.when(i == 0)
    def _():
        acc[...] = blk

    @pl.when(i > 0)
    def _():
        acc[...] = acc[...] + blk

    @pl.when(i == pl.num_programs(0) - 1)
    def _():
        out_ref[...] = acc[...]


def _bn_pool(t, st, g, be, bid):
    return pl.pallas_call(
        _bn_pool_body,
        grid=(GRID,),
        in_specs=[
            pl.BlockSpec((BN, D), lambda i: (i, 0)),
            pl.BlockSpec((2, D), lambda i: (0, 0)),
            pl.BlockSpec((1, D), lambda i: (0, 0)),
            pl.BlockSpec((1, D), lambda i: (0, 0)),
            pl.BlockSpec((1, 1, BN), lambda i: (i, 0, 0)),
        ],
        out_specs=pl.BlockSpec((NB, D), lambda i: (0, 0)),
        out_shape=jax.ShapeDtypeStruct((NB, D), jnp.float32),
        scratch_shapes=[pltpu.VMEM((NB, D), jnp.float32)],
    )(t, st, g, be, bid)


def _spectral_normalize(*Ws):
    def init(W):
        o = W.shape[0]
        u0 = jnp.full((1, o), 1.0 / np.sqrt(o), jnp.float32)
        return (u0, lax.dot_general(u0, W, (((1,), (0,)), ((), ()))))

    def step(W, u, v):
        v = lax.dot_general(u, W, (((1,), (0,)), ((), ())))
        v = v / (jnp.sqrt(jnp.sum(v * v)) + 1e-12)
        u = lax.dot_general(v, W, (((1,), (1,)), ((), ())))
        u = u / (jnp.sqrt(jnp.sum(u * u)) + 1e-12)
        return (u, v)

    def body(_, uvs):
        return tuple(step(W, u, v) for W, (u, v) in zip(Ws, uvs))

    uvs = lax.fori_loop(0, 30, body, tuple(init(W) for W in Ws))
    out = []
    for W, (u, v) in zip(Ws, uvs):
        wv = lax.dot_general(v, W, (((1,), (1,)), ((), ())))
        out.append(W / jnp.sum(u * wv))
    return out


def _sn_topo_body(topo_ref, wt_ref, bt_ref, wf1_ref, wf2_ref,
                  gt_ref, wf1n_ref, wf2n_ref):
    wt, wf1, wf2 = _spectral_normalize(wt_ref[...], wf1_ref[...], wf2_ref[...])
    gt_ref[...] = jnp.maximum(
        lax.dot_general(topo_ref[...], wt, (((1,), (1,)), ((), ())),
                        preferred_element_type=jnp.float32) + bt_ref[...], 0.0)
    wf1n_ref[...] = wf1
    wf2n_ref[...] = wf2


def _sn_topo(topo, wt, bt, wf1, wf2):
    return pl.pallas_call(
        _sn_topo_body,
        out_shape=[
            jax.ShapeDtypeStruct((NB, D), jnp.float32),
            jax.ShapeDtypeStruct(wf1.shape, jnp.float32),
            jax.ShapeDtypeStruct(wf2.shape, jnp.float32),
        ],
    )(topo, wt, bt, wf1, wf2)


def _head_body(gs_ref, gt_ref, wf1_ref, bf1_ref, wf2_ref, bf2_ref, out_ref):
    gcat = jnp.concatenate([gs_ref[...], gt_ref[...]], axis=1)
    z = lax.dot_general(gcat, wf1_ref[...], (((1,), (1,)), ((), ())),
                        preferred_element_type=jnp.float32) + bf1_ref[...]
    z = jnp.where(z > 0, z, jnp.exp(jnp.minimum(z, 0.0)) - 1.0)
    out_ref[...] = lax.dot_general(z, wf2_ref[...], (((1,), (1,)), ((), ())),
                                   preferred_element_type=jnp.float32) + bf2_ref[...]


def _head(gs, gt, wf1n, bf1, wf2n, bf2):
    return pl.pallas_call(
        _head_body,
        out_shape=jax.ShapeDtypeStruct((NB, 16), jnp.float32),
    )(gs, gt, wf1n, bf1, wf2n, bf2)


def kernel(x, edge_index, batch, topo_vec, W1a, b1a, g1, be1, W1b, b1b,
           W2a, b2a, g2, be2, Wt, bt, Wf1, bf1, Wf2, bf2):
    x0 = x[:, :HD]
    x1 = x[:, HD:]
    src = edge_index[0].astype(jnp.int32)
    dst = edge_index[1].astype(jnp.int32)
    srcr = jnp.concatenate([src, jnp.zeros((EP - E,), jnp.int32)]).reshape(NT, NCH, CH)
    pad_dst = N + jnp.arange(EP - E, dtype=jnp.int32) % NTRASH
    dstr = jnp.concatenate([dst, pad_dst]).reshape(NT, NCH, CH)

    gt, wf1n, wf2n = _sn_topo(topo_vec, Wt, bt.reshape(1, -1), Wf1, Wf2)
    seg0, seg1 = _sc_segment_sum(x0, x1, srcr, dstr)
    t1, st1 = _mm_stats(x0, x1, seg0, seg1, W1a, b1a.reshape(1, -1))
    h0, h1 = _bn_mm(t1, st1, g1.reshape(1, -1), be1.reshape(1, -1),
                    W1b, b1b.reshape(1, -1))
    s20, s21 = _sc_segment_sum(h0, h1, srcr, dstr)
    t2, st2 = _mm_stats(h0, h1, s20, s21, W2a, b2a.reshape(1, -1))
    gs = _bn_pool(t2, st2, g2.reshape(1, -1), be2.reshape(1, -1),
                  batch.astype(jnp.int32).reshape(GRID, 1, BN))
    return _head(gs, gt, wf1n, bf1.reshape(1, -1), wf2n, bf2.reshape(1, -1))
